# baseline (device time: 19943 ns/iter reference)
import jax
import jax.numpy as jnp
from jax import lax
from jax.experimental import pallas as pl
from jax.experimental.pallas import tpu as pltpu

N_DEV = 4
B, SQ, SKV, HQ, DH = 2, 256, 1024, 4, 64
S_LOC = SKV // N_DEV
HD = HQ * DH
D_MODEL = 512


def _mask_chip0():
    qi = lax.broadcasted_iota(jnp.int32, (SQ, S_LOC), 0)
    kj = lax.broadcasted_iota(jnp.int32, (SQ, S_LOC), 1)
    return (jnp.abs(qi - kj) <= 128) | (kj < 32) | (qi < 32)


def _mask_chip1_local():
    qi = lax.broadcasted_iota(jnp.int32, (SQ // 2, S_LOC), 0) + 128
    kj = lax.broadcasted_iota(jnp.int32, (SQ // 2, S_LOC), 1) + 256
    return kj - qi <= 128


def kernel(x, Wq, K_ext, V_ext, Wo):
    def body(x_ref, wq_ref, k_ref, v_ref, wo_ref, out_ref,
             xv, wqv, wov, kv, vv, ov,
             acc_s1, acc_r1, acc_s2, acc_r2,
             l_s1, l_r1, l_s2, l_r2,
             send_sems, recv_sems, in_sems, out_sems):
        my = lax.axis_index("i")
        left = lax.rem(my + (N_DEV - 1), N_DEV)
        right = lax.rem(my + 1, N_DEV)
        even = lax.rem(my, 2) == 0
        p1 = jnp.where(even, right, left)
        p2 = jnp.where(even, left, right)

        in_copies = []
        for i, (src, dst) in enumerate(
                [(x_ref, xv), (wq_ref, wqv), (k_ref, kv), (v_ref, vv),
                 (wo_ref, wov)]):
            c = pltpu.make_async_copy(src, dst, in_sems.at[i])
            c.start()
            in_copies.append(c)

        barrier = pltpu.get_barrier_semaphore()
        for nbr in (left, right):
            pl.semaphore_signal(barrier, inc=1, device_id=(nbr,),
                                device_id_type=pl.DeviceIdType.MESH)
        pl.semaphore_wait(barrier, 2)

        for c in in_copies[:4]:
            c.wait()

        wq_b = wqv[...].astype(jnp.bfloat16)
        k_loc = kv
        v_loc = vv

        q_all = (lax.dot_general(
            xv[...].astype(jnp.bfloat16).reshape(B * SQ, D_MODEL), wq_b,
            (((1,), (0,)), ((), ())),
            preferred_element_type=jnp.float32,
        ) * 0.125).astype(jnp.bfloat16).reshape(B, SQ, HD)

        def blocks_partial(b, blocks, cover_all):
            if not cover_all:
                acc_s1[b] = jnp.zeros((SQ, HD), jnp.bfloat16)
                l_s1[b] = jnp.zeros((SQ, HQ), jnp.float32)
            for r0, nr, m in blocks:
                rs = slice(r0, r0 + nr)
                for h in range(HQ):
                    sl = slice(h * DH, (h + 1) * DH)
                    s = lax.dot_general(
                        q_all[b, rs, sl], k_loc[b, :, sl],
                        (((1,), (1,)), ((), ())),
                        preferred_element_type=jnp.float32,
                    )
                    p = jnp.exp(s if m is None else jnp.where(m, s, -1e9))
                    l_s1[b, rs, h:h + 1] = jnp.sum(p, axis=1, keepdims=True)
                    acc_s1[b, rs, sl] = lax.dot_general(
                        p.astype(jnp.bfloat16), v_loc[b, :, sl],
                        (((1,), (0,)), ((), ())),
                        preferred_element_type=jnp.float32,
                    ).astype(jnp.bfloat16)

        def partial(b):
            @pl.when(my == 0)
            def _():
                blocks_partial(b, [(0, SQ, _mask_chip0())], True)

            @pl.when(my == 1)
            def _():
                blocks_partial(
                    b, [(0, 32, None), (128, 128, _mask_chip1_local())],
                    False)

            @pl.when(my >= 2)
            def _():
                blocks_partial(b, [(0, 32, None)], False)

        def exchange(src, dst, partner, sem_idx):
            r = pltpu.make_async_remote_copy(
                src_ref=src, dst_ref=dst,
                send_sem=send_sems.at[sem_idx], recv_sem=recv_sems.at[sem_idx],
                device_id=(partner,), device_id_type=pl.DeviceIdType.MESH,
            )
            r.start()
            return r

        def finalize(b, r2a, r2l, wo_b):
            r2a.wait_recv()
            r2l.wait_recv()
            acc_tot = (acc_s2[b] + acc_r2[b]).astype(jnp.float32)
            l_inv = 1.0 / (l_s2[b] + l_r2[b])
            parts = []
            for h in range(HQ):
                parts.append(acc_tot[:, h * DH:(h + 1) * DH] *
                             l_inv[:, h:h + 1])
            ctx_b = jnp.concatenate(parts, axis=1).astype(jnp.bfloat16)
            ov[b] = lax.dot_general(
                ctx_b, wo_b, (((1,), (0,)), ((), ())),
                preferred_element_type=jnp.float32,
            ).astype(jnp.bfloat16)
            c = pltpu.make_async_copy(ov.at[b], out_ref.at[b],
                                      out_sems.at[b])
            c.start()
            return c

        rdmas = []
        s1 = []
        for b in range(B):
            partial(b)
            ra = exchange(acc_s1.at[b], acc_r1.at[b], p1, 2 * b)
            rl = exchange(l_s1.at[b], l_r1.at[b], p1, 2 * b + 1)
            s1.append((ra, rl))
            rdmas += [ra, rl]
        s2 = []
        for b in range(B):
            ra, rl = s1[b]
            ra.wait_recv()
            rl.wait_recv()
            acc_s2[b] = acc_s1[b] + acc_r1[b]
            l_s2[b] = l_s1[b] + l_r1[b]
            ra2 = exchange(acc_s2.at[b], acc_r2.at[b], p2, 4 + 2 * b)
            rl2 = exchange(l_s2.at[b], l_r2.at[b], p2, 4 + 2 * b + 1)
            s2.append((ra2, rl2))
            rdmas += [ra2, rl2]
        in_copies[4].wait()
        wo_b = wov[...].astype(jnp.bfloat16)
        out_copies = []
        for b in range(B):
            out_copies.append(finalize(b, *s2[b], wo_b))

        for c in out_copies:
            c.wait()
        for r in rdmas:
            r.wait_send()

    K_ext = K_ext.astype(jnp.bfloat16).reshape(B, S_LOC, HD)
    V_ext = V_ext.astype(jnp.bfloat16).reshape(B, S_LOC, HD)

    return pl.pallas_call(
        body,
        out_shape=jax.ShapeDtypeStruct((B, SQ, D_MODEL), jnp.bfloat16),
        in_specs=[pl.BlockSpec(memory_space=pl.ANY)] * 5,
        out_specs=pl.BlockSpec(memory_space=pl.ANY),
        scratch_shapes=[
            pltpu.VMEM((B, SQ, D_MODEL), jnp.float32),
            pltpu.VMEM((D_MODEL, HD), jnp.float32),
            pltpu.VMEM((HD, D_MODEL), jnp.float32),
            pltpu.VMEM((B, S_LOC, HD), jnp.bfloat16),
            pltpu.VMEM((B, S_LOC, HD), jnp.bfloat16),
            pltpu.VMEM((B, SQ, D_MODEL), jnp.bfloat16),
            pltpu.VMEM((B, SQ, HD), jnp.bfloat16),
            pltpu.VMEM((B, SQ, HD), jnp.bfloat16),
            pltpu.VMEM((B, SQ, HD), jnp.bfloat16),
            pltpu.VMEM((B, SQ, HD), jnp.bfloat16),
            pltpu.VMEM((B, SQ, HQ), jnp.float32),
            pltpu.VMEM((B, SQ, HQ), jnp.float32),
            pltpu.VMEM((B, SQ, HQ), jnp.float32),
            pltpu.VMEM((B, SQ, HQ), jnp.float32),
            pltpu.SemaphoreType.DMA((8,)),
            pltpu.SemaphoreType.DMA((8,)),
            pltpu.SemaphoreType.DMA((5,)),
            pltpu.SemaphoreType.DMA((2,)),
        ],
        compiler_params=pltpu.CompilerParams(collective_id=0),
    )(x, Wq, K_ext, V_ext, Wo)


# device time: 19737 ns/iter; 1.0104x vs baseline; 1.0104x over previous
import jax
import jax.numpy as jnp
from jax import lax
from jax.experimental import pallas as pl
from jax.experimental.pallas import tpu as pltpu

N_DEV = 4
B, SQ, SKV, HQ, DH = 2, 256, 1024, 4, 64
S_LOC = SKV // N_DEV
HD = HQ * DH
D_MODEL = 512


def _mask_chip0():
    qi = lax.broadcasted_iota(jnp.int32, (SQ, S_LOC), 0)
    kj = lax.broadcasted_iota(jnp.int32, (SQ, S_LOC), 1)
    return (jnp.abs(qi - kj) <= 128) | (kj < 32) | (qi < 32)


def _mask_chip1_local():
    qi = lax.broadcasted_iota(jnp.int32, (SQ // 2, S_LOC), 0) + 128
    kj = lax.broadcasted_iota(jnp.int32, (SQ // 2, S_LOC), 1) + 256
    return kj - qi <= 128


def kernel(x, Wq, K_ext, V_ext, Wo):
    def body(x_ref, wq_ref, k_ref, v_ref, wo_ref, out_ref,
             acc_s1, acc_r1, acc_s2, acc_r2,
             l_s1, l_r1, l_s2, l_r2,
             send_sems, recv_sems):
        my = lax.axis_index("i")
        left = lax.rem(my + (N_DEV - 1), N_DEV)
        right = lax.rem(my + 1, N_DEV)
        even = lax.rem(my, 2) == 0
        p1 = jnp.where(even, right, left)
        p2 = jnp.where(even, left, right)

        barrier = pltpu.get_barrier_semaphore()
        for nbr in (left, right):
            pl.semaphore_signal(barrier, inc=1, device_id=(nbr,),
                                device_id_type=pl.DeviceIdType.MESH)
        pl.semaphore_wait(barrier, 2)

        wq_b = wq_ref[...].astype(jnp.bfloat16)
        wo_b = wo_ref[...].astype(jnp.bfloat16)
        k_loc = k_ref
        v_loc = v_ref

        q_all = (lax.dot_general(
            x_ref[...].astype(jnp.bfloat16).reshape(B * SQ, D_MODEL), wq_b,
            (((1,), (0,)), ((), ())),
            preferred_element_type=jnp.float32,
        ) * 0.125).astype(jnp.bfloat16).reshape(B, SQ, HD)

        def blocks_partial(b, blocks, cover_all):
            if not cover_all:
                acc_s1[b] = jnp.zeros((SQ, HD), jnp.bfloat16)
                l_s1[b] = jnp.zeros((SQ, HQ), jnp.float32)
            for r0, nr, m in blocks:
                rs = slice(r0, r0 + nr)
                for h in range(HQ):
                    sl = slice(h * DH, (h + 1) * DH)
                    s = lax.dot_general(
                        q_all[b, rs, sl], k_loc[b, :, sl],
                        (((1,), (1,)), ((), ())),
                        preferred_element_type=jnp.float32,
                    )
                    p = jnp.exp(s if m is None else jnp.where(m, s, -1e9))
                    l_s1[b, rs, h:h + 1] = jnp.sum(p, axis=1, keepdims=True)
                    acc_s1[b, rs, sl] = lax.dot_general(
                        p.astype(jnp.bfloat16), v_loc[b, :, sl],
                        (((1,), (0,)), ((), ())),
                        preferred_element_type=jnp.float32,
                    ).astype(jnp.bfloat16)

        def partial(b):
            @pl.when(my == 0)
            def _():
                blocks_partial(b, [(0, SQ, _mask_chip0())], True)

            @pl.when(my == 1)
            def _():
                blocks_partial(
                    b, [(0, 32, None), (128, 128, _mask_chip1_local())],
                    False)

            @pl.when(my >= 2)
            def _():
                blocks_partial(b, [(0, 32, None)], False)

        def exchange(src, dst, partner, sem_idx):
            r = pltpu.make_async_remote_copy(
                src_ref=src, dst_ref=dst,
                send_sem=send_sems.at[sem_idx], recv_sem=recv_sems.at[sem_idx],
                device_id=(partner,), device_id_type=pl.DeviceIdType.MESH,
            )
            r.start()
            return r

        def finalize(b, r2a, r2l):
            r2a.wait_recv()
            r2l.wait_recv()
            acc_tot = (acc_s2[b] + acc_r2[b]).astype(jnp.float32)
            l_inv = 1.0 / (l_s2[b] + l_r2[b])
            parts = []
            for h in range(HQ):
                parts.append(acc_tot[:, h * DH:(h + 1) * DH] *
                             l_inv[:, h:h + 1])
            ctx_b = jnp.concatenate(parts, axis=1).astype(jnp.bfloat16)
            out_ref[b] = lax.dot_general(
                ctx_b, wo_b, (((1,), (0,)), ((), ())),
                preferred_element_type=jnp.float32,
            ).astype(jnp.bfloat16)

        rdmas = []
        s1 = []
        for b in range(B):
            partial(b)
            ra = exchange(acc_s1.at[b], acc_r1.at[b], p1, 2 * b)
            rl = exchange(l_s1.at[b], l_r1.at[b], p1, 2 * b + 1)
            s1.append((ra, rl))
            rdmas += [ra, rl]
        s2 = []
        for b in range(B):
            ra, rl = s1[b]
            ra.wait_recv()
            rl.wait_recv()
            acc_s2[b] = acc_s1[b] + acc_r1[b]
            l_s2[b] = l_s1[b] + l_r1[b]
            ra2 = exchange(acc_s2.at[b], acc_r2.at[b], p2, 4 + 2 * b)
            rl2 = exchange(l_s2.at[b], l_r2.at[b], p2, 4 + 2 * b + 1)
            s2.append((ra2, rl2))
            rdmas += [ra2, rl2]
        for b in range(B):
            finalize(b, *s2[b])

        for r in rdmas:
            r.wait_send()

    K_ext = K_ext.astype(jnp.bfloat16).reshape(B, S_LOC, HD)
    V_ext = V_ext.astype(jnp.bfloat16).reshape(B, S_LOC, HD)

    return pl.pallas_call(
        body,
        out_shape=jax.ShapeDtypeStruct((B, SQ, D_MODEL), jnp.bfloat16),
        in_specs=[pl.BlockSpec(memory_space=pltpu.VMEM)] * 5,
        out_specs=pl.BlockSpec(memory_space=pltpu.VMEM),
        scratch_shapes=[
            pltpu.VMEM((B, SQ, HD), jnp.bfloat16),
            pltpu.VMEM((B, SQ, HD), jnp.bfloat16),
            pltpu.VMEM((B, SQ, HD), jnp.bfloat16),
            pltpu.VMEM((B, SQ, HD), jnp.bfloat16),
            pltpu.VMEM((B, SQ, HQ), jnp.float32),
            pltpu.VMEM((B, SQ, HQ), jnp.float32),
            pltpu.VMEM((B, SQ, HQ), jnp.float32),
            pltpu.VMEM((B, SQ, HQ), jnp.float32),
            pltpu.SemaphoreType.DMA((8,)),
            pltpu.SemaphoreType.DMA((8,)),
        ],
        compiler_params=pltpu.CompilerParams(collective_id=0),
    )(x, Wq, K_ext, V_ext, Wo)
